# baseline (speedup 1.0000x reference)
import functools
import jax
import jax.numpy as jnp
from jax import lax
from jax.experimental import pallas as pl
from jax.experimental.pallas import tpu as pltpu
from jax.experimental.pallas import tpu_sc as plsc

_N = 10000
_D = 128
_G = 16
_E = 320000

_NC = 2
_NS = 16
_NW = _NC * _NS
_CHUNK = 128
_PH = 2
_C2 = 40
_C = _PH * _C2
_E_PAD = _NW * _CHUNK * _C
_ROWS = 10240
_STRIPE = _ROWS // _NS
_DUMMY = _N + 8


def _sc_agg(x, src_idx, dst_idx):
    mesh = plsc.VectorSubcoreMesh(core_axis_name="c", subcore_axis_name="s")

    @functools.partial(
        pl.kernel,
        out_type=jax.ShapeDtypeStruct((_NC, _ROWS, _D), jnp.float32),
        mesh=mesh,
        scratch_types=[
            pltpu.VMEM((_C2, _CHUNK), jnp.int32),
            pltpu.VMEM((_C2, _CHUNK), jnp.int32),
            pltpu.VMEM((_CHUNK, _D), jnp.float32),
            pltpu.VMEM((_CHUNK, _D), jnp.float32),
            pltpu.VMEM_SHARED((_ROWS, _D), jnp.float32),
            pltpu.SemaphoreType.DMA,
            pltpu.SemaphoreType.DMA,
        ],
    )
    def k(x_hbm, src_hbm, dst_hbm, out_hbm, src_v, dst_v, rows_v, rows2_v,
          acc_sh, sem, sem2):
        cid = lax.axis_index("c")
        sid = lax.axis_index("s")
        wid = sid * _NC + cid

        zero16 = jnp.zeros((16,), jnp.float32)

        def zb(i, carry):
            rows_v[i // (_D // 16), pl.ds((i % (_D // 16)) * 16, 16)] = zero16
            return carry

        lax.fori_loop(0, _CHUNK * (_D // 16), zb, 0)
        r0 = sid * _STRIPE
        for t in range(_STRIPE // _CHUNK):
            pltpu.sync_copy(rows_v, acc_sh.at[pl.ds(r0 + t * _CHUNK, _CHUNK)])

        plsc.subcore_barrier()

        for ph in range(_PH):
            pltpu.sync_copy(src_hbm.at[wid, ph], src_v)
            pltpu.sync_copy(dst_hbm.at[wid, ph], dst_v)
            pltpu.async_copy(x_hbm.at[src_v.at[0]], rows_v, sem)

            def body(k, carry):
                j0 = 2 * k
                j1 = j0 + 1
                pltpu.make_async_copy(x_hbm.at[src_v.at[j0]], rows_v, sem).wait()
                pltpu.async_copy(x_hbm.at[src_v.at[j1]], rows2_v, sem2)
                pltpu.sync_copy(rows_v, acc_sh.at[dst_v.at[j0]], add=True)
                pltpu.make_async_copy(x_hbm.at[src_v.at[j1]], rows2_v, sem2).wait()
                pltpu.async_copy(x_hbm.at[src_v.at[j0 + 2]], rows_v, sem)
                pltpu.sync_copy(rows2_v, acc_sh.at[dst_v.at[j1]], add=True)
                return carry

            lax.fori_loop(0, _C2 // 2 - 1, body, 0)
            j0 = _C2 - 2
            pltpu.make_async_copy(x_hbm.at[src_v.at[j0]], rows_v, sem).wait()
            pltpu.async_copy(x_hbm.at[src_v.at[j0 + 1]], rows2_v, sem2)
            pltpu.sync_copy(rows_v, acc_sh.at[dst_v.at[j0]], add=True)
            pltpu.make_async_copy(x_hbm.at[src_v.at[j0 + 1]], rows2_v, sem2).wait()
            pltpu.sync_copy(rows2_v, acc_sh.at[dst_v.at[j0 + 1]], add=True)
        plsc.subcore_barrier()

        for t in range(_STRIPE // _CHUNK):
            rr = r0 + t * _CHUNK
            pltpu.sync_copy(acc_sh.at[pl.ds(rr, _CHUNK)], rows_v)
            pltpu.sync_copy(rows_v, out_hbm.at[cid, pl.ds(rr, _CHUNK)])

    return k(x, src_idx, dst_idx)


_BN = 1000


def _tc_body(x_ref, a0_ref, a1_ref, ew_ref, bt_ref, w1_ref, b1_ref, w2_ref,
             b2_ref, out_ref):
    h = x_ref[...] + a0_ref[...] + a1_ref[...]
    h = jnp.dot(h, w1_ref[...], preferred_element_type=jnp.float32) + b1_ref[...]
    h = jnp.maximum(h, 0.0)
    h = jnp.dot(h, w2_ref[...], preferred_element_type=jnp.float32) + b2_ref[...]
    bt = bt_ref[0, 0, :]
    ew = ew_ref[0, 0, :]
    gids = lax.broadcasted_iota(jnp.int32, (_G, _BN), 0)
    wgt = jnp.where(bt[None, :] == gids, ew[None, :], 0.0)
    contrib = jnp.dot(wgt, h, preferred_element_type=jnp.float32)

    @pl.when(pl.program_id(0) == 0)
    def _():
        out_ref[...] = jnp.zeros_like(out_ref)

    out_ref[...] += contrib


def _tc_mlp_pool(x, a0, a1, ew, bt, W1, b1, W2, b2):
    nb = _N // _BN
    return pl.pallas_call(
        _tc_body,
        grid=(nb,),
        in_specs=[
            pl.BlockSpec((_BN, _D), lambda i: (i, 0)),
            pl.BlockSpec((_BN, _D), lambda i: (i, 0)),
            pl.BlockSpec((_BN, _D), lambda i: (i, 0)),
            pl.BlockSpec((1, 1, _BN), lambda i: (i, 0, 0)),
            pl.BlockSpec((1, 1, _BN), lambda i: (i, 0, 0)),
            pl.BlockSpec((_D, _D), lambda i: (0, 0)),
            pl.BlockSpec((1, _D), lambda i: (0, 0)),
            pl.BlockSpec((_D, _D), lambda i: (0, 0)),
            pl.BlockSpec((1, _D), lambda i: (0, 0)),
        ],
        out_specs=pl.BlockSpec((_G, _D), lambda i: (0, 0)),
        out_shape=jax.ShapeDtypeStruct((_G, _D), jnp.float32),
    )(x, a0, a1, ew, bt, W1, b1, W2, b2)


def kernel(x, p, edge_index, batch, W1, b1, W2, b2):
    src = edge_index[0]
    dst = edge_index[1]
    pad = _E_PAD - _E
    srcp = jnp.concatenate([src, jnp.zeros((pad,), jnp.int32)])
    dstp = jnp.concatenate([dst, jnp.full((pad,), _DUMMY, jnp.int32)])
    src_idx = srcp.reshape(_NW, _PH, _C2, _CHUNK)
    dst_idx = dstp.reshape(_NW, _PH, _C2, _CHUNK)

    parts = _sc_agg(x, src_idx, dst_idx)

    nb = _N // _BN
    ew = p[:, 0].reshape(nb, 1, _BN)
    bt = batch.reshape(nb, 1, _BN)
    return _tc_mlp_pool(x, parts[0, :_N], parts[1, :_N], ew, bt,
                        W1, b1.reshape(1, _D), W2, b2.reshape(1, _D))


# speedup vs baseline: 1.0001x; 1.0001x over previous
import functools
import jax
import jax.numpy as jnp
from jax import lax
from jax.experimental import pallas as pl
from jax.experimental.pallas import tpu as pltpu
from jax.experimental.pallas import tpu_sc as plsc

_N = 10000
_D = 128
_G = 16
_E = 320000

_NC = 2
_NS = 16
_NW = _NC * _NS
_CHUNK = 128
_PH = 2
_C2 = 40
_C = _PH * _C2
_E_PAD = _NW * _CHUNK * _C
_ROWS = 10240
_STRIPE = _ROWS // _NS
_DUMMY = _N + 8


def _sc_agg(x, src_idx, dst_idx):
    mesh = plsc.VectorSubcoreMesh(core_axis_name="c", subcore_axis_name="s")

    @functools.partial(
        pl.kernel,
        out_type=jax.ShapeDtypeStruct((_NC, _ROWS, _D), jnp.float32),
        mesh=mesh,
        scratch_types=[
            pltpu.VMEM((_C2, _CHUNK), jnp.int32),
            pltpu.VMEM((_C2, _CHUNK), jnp.int32),
            pltpu.VMEM((_CHUNK, _D), jnp.float32),
            pltpu.VMEM((_CHUNK, _D), jnp.float32),
            pltpu.VMEM_SHARED((_ROWS, _D), jnp.float32),
            pltpu.SemaphoreType.DMA,
            pltpu.SemaphoreType.DMA,
        ],
    )
    def k(x_hbm, src_hbm, dst_hbm, out_hbm, src_v, dst_v, rows_v, rows2_v,
          acc_sh, sem, sem2):
        cid = lax.axis_index("c")
        sid = lax.axis_index("s")
        wid = sid * _NC + cid

        zero16 = jnp.zeros((16,), jnp.float32)

        def zb(i, carry):
            rows_v[i // (_D // 16), pl.ds((i % (_D // 16)) * 16, 16)] = zero16
            return carry

        lax.fori_loop(0, _CHUNK * (_D // 16), zb, 0)
        r0 = sid * _STRIPE
        for t in range(_STRIPE // _CHUNK):
            pltpu.sync_copy(rows_v, acc_sh.at[pl.ds(r0 + t * _CHUNK, _CHUNK)])

        plsc.subcore_barrier()

        for ph in range(_PH):
            pltpu.sync_copy(src_hbm.at[wid, ph], src_v)
            pltpu.sync_copy(dst_hbm.at[wid, ph], dst_v)
            pltpu.async_copy(x_hbm.at[src_v.at[0]], rows_v, sem)

            def body(k, carry):
                j0 = 2 * k
                j1 = j0 + 1
                pltpu.make_async_copy(x_hbm.at[src_v.at[j0]], rows_v, sem).wait()
                pltpu.async_copy(x_hbm.at[src_v.at[j1]], rows2_v, sem2)
                pltpu.sync_copy(rows_v, acc_sh.at[dst_v.at[j0]], add=True)
                pltpu.make_async_copy(x_hbm.at[src_v.at[j1]], rows2_v, sem2).wait()
                pltpu.async_copy(x_hbm.at[src_v.at[j0 + 2]], rows_v, sem)
                pltpu.sync_copy(rows2_v, acc_sh.at[dst_v.at[j1]], add=True)
                return carry

            lax.fori_loop(0, _C2 // 2 - 1, body, 0)
            j0 = _C2 - 2
            pltpu.make_async_copy(x_hbm.at[src_v.at[j0]], rows_v, sem).wait()
            pltpu.async_copy(x_hbm.at[src_v.at[j0 + 1]], rows2_v, sem2)
            pltpu.sync_copy(rows_v, acc_sh.at[dst_v.at[j0]], add=True)
            pltpu.make_async_copy(x_hbm.at[src_v.at[j0 + 1]], rows2_v, sem2).wait()
            pltpu.sync_copy(rows2_v, acc_sh.at[dst_v.at[j0 + 1]], add=True)
        plsc.subcore_barrier()

        for t in range(_STRIPE // _CHUNK):
            rr = r0 + t * _CHUNK
            pltpu.sync_copy(acc_sh.at[pl.ds(rr, _CHUNK)], rows_v)
            pltpu.sync_copy(rows_v, out_hbm.at[cid, pl.ds(rr, _CHUNK)])

    return k(x, src_idx, dst_idx)


_BN = 1000


def _tc_body(x_ref, a0_ref, a1_ref, ew_ref, bt_ref, w1_ref, b1_ref, w2_ref,
             b2_ref, out_ref):
    h = x_ref[...] + a0_ref[...] + a1_ref[...]
    h = jnp.dot(h, w1_ref[...], preferred_element_type=jnp.float32) + b1_ref[...]
    h = jnp.maximum(h, 0.0)
    h = jnp.dot(h, w2_ref[...], preferred_element_type=jnp.float32) + b2_ref[...]
    bt = bt_ref[0, 0, :]
    ew = ew_ref[0, 0, :]
    gids = lax.broadcasted_iota(jnp.int32, (_G, _BN), 0)
    wgt = jnp.where(bt[None, :] == gids, ew[None, :], 0.0)
    contrib = jnp.dot(wgt, h, preferred_element_type=jnp.float32)

    @pl.when(pl.program_id(0) == 0)
    def _():
        out_ref[...] = jnp.zeros_like(out_ref)

    out_ref[...] += contrib


def _tc_mlp_pool(x, a0, a1, ew, bt, W1, b1, W2, b2):
    nb = _N // _BN
    return pl.pallas_call(
        _tc_body,
        grid=(nb,),
        in_specs=[
            pl.BlockSpec((_BN, _D), lambda i: (i, 0)),
            pl.BlockSpec((_BN, _D), lambda i: (i, 0)),
            pl.BlockSpec((_BN, _D), lambda i: (i, 0)),
            pl.BlockSpec((1, 1, _BN), lambda i: (i, 0, 0)),
            pl.BlockSpec((1, 1, _BN), lambda i: (i, 0, 0)),
            pl.BlockSpec((_D, _D), lambda i: (0, 0)),
            pl.BlockSpec((1, _D), lambda i: (0, 0)),
            pl.BlockSpec((_D, _D), lambda i: (0, 0)),
            pl.BlockSpec((1, _D), lambda i: (0, 0)),
        ],
        out_specs=pl.BlockSpec((_G, _D), lambda i: (0, 0)),
        out_shape=jax.ShapeDtypeStruct((_G, _D), jnp.float32),
    )(x, a0, a1, ew, bt, W1, b1, W2, b2)


def kernel(x, p, edge_index, batch, W1, b1, W2, b2):
    src = edge_index[0]
    dst = edge_index[1]
    pad = _E_PAD - _E
    srcp = jnp.concatenate([src, jnp.zeros((pad,), jnp.int32)])
    dummy = _N + jnp.arange(pad, dtype=jnp.int32) % (_ROWS - _N)
    dstp = jnp.concatenate([dst, dummy])
    src_idx = srcp.reshape(_NW, _PH, _C2, _CHUNK)
    dst_idx = dstp.reshape(_NW, _PH, _C2, _CHUNK)

    parts = _sc_agg(x, src_idx, dst_idx)

    nb = _N // _BN
    ew = p[:, 0].reshape(nb, 1, _BN)
    bt = batch.reshape(nb, 1, _BN)
    return _tc_mlp_pool(x, parts[0, :_N], parts[1, :_N], ew, bt,
                        W1, b1.reshape(1, _D), W2, b2.reshape(1, _D))


# speedup vs baseline: 1.3831x; 1.3830x over previous
import functools
import jax
import jax.numpy as jnp
from jax import lax
from jax.experimental import pallas as pl
from jax.experimental.pallas import tpu as pltpu
from jax.experimental.pallas import tpu_sc as plsc

_N = 10000
_D = 128
_G = 16
_E = 320000

_NC = 2
_NS = 16
_NW = _NC * _NS
_CHUNK = 128
_C = -(-_E // (_NW * _CHUNK))
_E_PAD = _NW * _CHUNK * _C
_ROWS = 10112
_STRIPE = _ROWS // _NS
_WB = (128, 128, 128, 128, 120)


def _sc_agg(x, src_idx, dst_idx):
    mesh = plsc.VectorSubcoreMesh(core_axis_name="c", subcore_axis_name="s")

    @functools.partial(
        pl.kernel,
        out_type=jax.ShapeDtypeStruct((_NC, _ROWS, _D), jnp.float32),
        mesh=mesh,
        scratch_types=[
            pltpu.VMEM((_C, _CHUNK), jnp.int32),
            pltpu.VMEM((_C, _CHUNK), jnp.int32),
            pltpu.VMEM((_CHUNK, _D), jnp.float32),
            pltpu.VMEM_SHARED((_ROWS, _D), jnp.float32),
            pltpu.SemaphoreType.DMA,
        ],
    )
    def k(x_hbm, src_hbm, dst_hbm, out_hbm, src_v, dst_v, rows_v, acc_sh, sem):
        cid = lax.axis_index("c")
        sid = lax.axis_index("s")
        wid = sid * _NC + cid

        zero16 = jnp.zeros((16,), jnp.float32)

        def zb(i, carry):
            rows_v[i // (_D // 16), pl.ds((i % (_D // 16)) * 16, 16)] = zero16
            return carry

        lax.fori_loop(0, _CHUNK * (_D // 16), zb, 0)
        r0 = sid * _STRIPE
        off = 0
        for w in _WB:
            pltpu.sync_copy(rows_v.at[pl.ds(0, w)],
                            acc_sh.at[pl.ds(r0 + off, w)])
            off += w

        pltpu.sync_copy(src_hbm.at[wid], src_v)
        pltpu.sync_copy(dst_hbm.at[wid], dst_v)
        plsc.subcore_barrier()

        def body(j, carry):
            pltpu.async_copy(x_hbm.at[src_v.at[j]], rows_v, sem).wait()
            pltpu.sync_copy(rows_v, acc_sh.at[dst_v.at[j]], add=True)
            return carry

        lax.fori_loop(0, _C, body, 0)
        plsc.subcore_barrier()

        off = 0
        for w in _WB:
            rr = r0 + off
            pltpu.sync_copy(acc_sh.at[pl.ds(rr, w)], rows_v.at[pl.ds(0, w)])
            pltpu.sync_copy(rows_v.at[pl.ds(0, w)], out_hbm.at[cid, pl.ds(rr, w)])
            off += w

    return k(x, src_idx, dst_idx)


_BN = 1000


def _tc_body(x_ref, a0_ref, a1_ref, ew_ref, bt_ref, w1_ref, b1_ref, w2_ref,
             b2_ref, out_ref):
    h = x_ref[...] + a0_ref[...] + a1_ref[...]
    h = jnp.dot(h, w1_ref[...], preferred_element_type=jnp.float32) + b1_ref[...]
    h = jnp.maximum(h, 0.0)
    h = jnp.dot(h, w2_ref[...], preferred_element_type=jnp.float32) + b2_ref[...]
    bt = bt_ref[0, 0, :]
    ew = ew_ref[0, 0, :]
    gids = lax.broadcasted_iota(jnp.int32, (_G, _BN), 0)
    wgt = jnp.where(bt[None, :] == gids, ew[None, :], 0.0)
    contrib = jnp.dot(wgt, h, preferred_element_type=jnp.float32)

    @pl.when(pl.program_id(0) == 0)
    def _():
        out_ref[...] = jnp.zeros_like(out_ref)

    out_ref[...] += contrib


def _tc_mlp_pool(x, a0, a1, ew, bt, W1, b1, W2, b2):
    nb = _N // _BN
    return pl.pallas_call(
        _tc_body,
        grid=(nb,),
        in_specs=[
            pl.BlockSpec((_BN, _D), lambda i: (i, 0)),
            pl.BlockSpec((_BN, _D), lambda i: (i, 0)),
            pl.BlockSpec((_BN, _D), lambda i: (i, 0)),
            pl.BlockSpec((1, 1, _BN), lambda i: (i, 0, 0)),
            pl.BlockSpec((1, 1, _BN), lambda i: (i, 0, 0)),
            pl.BlockSpec((_D, _D), lambda i: (0, 0)),
            pl.BlockSpec((1, _D), lambda i: (0, 0)),
            pl.BlockSpec((_D, _D), lambda i: (0, 0)),
            pl.BlockSpec((1, _D), lambda i: (0, 0)),
        ],
        out_specs=pl.BlockSpec((_G, _D), lambda i: (0, 0)),
        out_shape=jax.ShapeDtypeStruct((_G, _D), jnp.float32),
    )(x, a0, a1, ew, bt, W1, b1, W2, b2)


def kernel(x, p, edge_index, batch, W1, b1, W2, b2):
    src = edge_index[0]
    dst = edge_index[1]
    pad = _E_PAD - _E
    srcp = jnp.concatenate([src, jnp.zeros((pad,), jnp.int32)])
    dummy = _N + jnp.arange(pad, dtype=jnp.int32) % (_ROWS - _N)
    dstp = jnp.concatenate([dst, dummy])
    src_idx = srcp.reshape(_NW, _C, _CHUNK)
    dst_idx = dstp.reshape(_NW, _C, _CHUNK)

    parts = _sc_agg(x, src_idx, dst_idx)

    nb = _N // _BN
    ew = p[:, 0].reshape(nb, 1, _BN)
    bt = batch.reshape(nb, 1, _BN)
    return _tc_mlp_pool(x, parts[0, :_N], parts[1, :_N], ew, bt,
                        W1, b1.reshape(1, _D), W2, b2.reshape(1, _D))


# speedup vs baseline: 1.5440x; 1.1164x over previous
import functools
import jax
import jax.numpy as jnp
from jax import lax
from jax.experimental import pallas as pl
from jax.experimental.pallas import tpu as pltpu
from jax.experimental.pallas import tpu_sc as plsc

_N = 10000
_D = 128
_G = 16
_E = 320000

_NC = 2
_NS = 16
_NW = _NC * _NS
_CHUNK = 112
_C = -(-_E // (_NW * _CHUNK))
_E_PAD = _NW * _CHUNK * _C
_ROWS = 10112
_STRIPE = _ROWS // _NS
_WB = (128, 128, 128, 128, 120)


def _sc_agg(x, src_idx, dst_idx):
    mesh = plsc.VectorSubcoreMesh(core_axis_name="c", subcore_axis_name="s")

    @functools.partial(
        pl.kernel,
        out_type=jax.ShapeDtypeStruct((_NC, _ROWS, _D), jnp.float32),
        mesh=mesh,
        scratch_types=[
            pltpu.VMEM((_C, _CHUNK), jnp.int32),
            pltpu.VMEM((_C, _CHUNK), jnp.int32),
            pltpu.VMEM((_CHUNK, _D), jnp.float32),
            pltpu.VMEM_SHARED((_ROWS, _D), jnp.float32),
            pltpu.SemaphoreType.DMA,
        ],
    )
    def k(x_hbm, src_hbm, dst_hbm, out_hbm, src_v, dst_v, rows_v, acc_sh, sem):
        cid = lax.axis_index("c")
        sid = lax.axis_index("s")
        wid = sid * _NC + cid

        zero16 = jnp.zeros((16,), jnp.float32)

        def zb(i, carry):
            rows_v[i // (_D // 16), pl.ds((i % (_D // 16)) * 16, 16)] = zero16
            return carry

        lax.fori_loop(0, _CHUNK * (_D // 16), zb, 0)
        r0 = sid * _STRIPE
        off = 0
        for w in _WB:
            pltpu.sync_copy(rows_v.at[pl.ds(0, w)],
                            acc_sh.at[pl.ds(r0 + off, w)])
            off += w

        pltpu.sync_copy(src_hbm.at[wid], src_v)
        pltpu.sync_copy(dst_hbm.at[wid], dst_v)
        plsc.subcore_barrier()

        def body(j, carry):
            pltpu.async_copy(x_hbm.at[src_v.at[j]], rows_v, sem).wait()
            pltpu.sync_copy(rows_v, acc_sh.at[dst_v.at[j]], add=True)
            return carry

        lax.fori_loop(0, _C, body, 0)
        plsc.subcore_barrier()

        off = 0
        for w in _WB:
            rr = r0 + off
            pltpu.sync_copy(acc_sh.at[pl.ds(rr, w)], rows_v.at[pl.ds(0, w)])
            pltpu.sync_copy(rows_v.at[pl.ds(0, w)], out_hbm.at[cid, pl.ds(rr, w)])
            off += w

    return k(x, src_idx, dst_idx)


_BN = 1000


def _tc_body(x_ref, a0_ref, a1_ref, ew_ref, bt_ref, w1_ref, b1_ref, w2_ref,
             b2_ref, out_ref):
    h = x_ref[...] + a0_ref[...] + a1_ref[...]
    h = jnp.dot(h, w1_ref[...], preferred_element_type=jnp.float32) + b1_ref[...]
    h = jnp.maximum(h, 0.0)
    h = jnp.dot(h, w2_ref[...], preferred_element_type=jnp.float32) + b2_ref[...]
    bt = bt_ref[0, 0, :]
    ew = ew_ref[0, 0, :]
    gids = lax.broadcasted_iota(jnp.int32, (_G, _BN), 0)
    wgt = jnp.where(bt[None, :] == gids, ew[None, :], 0.0)
    contrib = jnp.dot(wgt, h, preferred_element_type=jnp.float32)

    @pl.when(pl.program_id(0) == 0)
    def _():
        out_ref[...] = jnp.zeros_like(out_ref)

    out_ref[...] += contrib


def _tc_mlp_pool(x, a0, a1, ew, bt, W1, b1, W2, b2):
    nb = _N // _BN
    return pl.pallas_call(
        _tc_body,
        grid=(nb,),
        in_specs=[
            pl.BlockSpec((_BN, _D), lambda i: (i, 0)),
            pl.BlockSpec((_BN, _D), lambda i: (i, 0)),
            pl.BlockSpec((_BN, _D), lambda i: (i, 0)),
            pl.BlockSpec((1, 1, _BN), lambda i: (i, 0, 0)),
            pl.BlockSpec((1, 1, _BN), lambda i: (i, 0, 0)),
            pl.BlockSpec((_D, _D), lambda i: (0, 0)),
            pl.BlockSpec((1, _D), lambda i: (0, 0)),
            pl.BlockSpec((_D, _D), lambda i: (0, 0)),
            pl.BlockSpec((1, _D), lambda i: (0, 0)),
        ],
        out_specs=pl.BlockSpec((_G, _D), lambda i: (0, 0)),
        out_shape=jax.ShapeDtypeStruct((_G, _D), jnp.float32),
    )(x, a0, a1, ew, bt, W1, b1, W2, b2)


def kernel(x, p, edge_index, batch, W1, b1, W2, b2):
    src = edge_index[0]
    dst = edge_index[1]
    pad = _E_PAD - _E
    srcp = jnp.concatenate([src, jnp.zeros((pad,), jnp.int32)])
    dummy = _N + jnp.arange(pad, dtype=jnp.int32) % (_ROWS - _N)
    dstp = jnp.concatenate([dst, dummy])
    src_idx = srcp.reshape(_NW, _C, _CHUNK)
    dst_idx = dstp.reshape(_NW, _C, _CHUNK)

    parts = _sc_agg(x, src_idx, dst_idx)

    nb = _N // _BN
    ew = p[:, 0].reshape(nb, 1, _BN)
    bt = batch.reshape(nb, 1, _BN)
    return _tc_mlp_pool(x, parts[0, :_N], parts[1, :_N], ew, bt,
                        W1, b1.reshape(1, _D), W2, b2.reshape(1, _D))
